# Initial kernel scaffold; baseline (speedup 1.0000x reference)
import jax
import jax.numpy as jnp
from jax.experimental import pallas as pl


def kernel(node_features, edge_features, senders, receivers, nW1, nb1, nW2, nb2, ng, nbeta, eW1, eb1, eW2, eb2, eg, ebeta):
    raise NotImplementedError("write your pallas kernel here")


# speedup vs baseline: 3.1950x; 3.1950x over previous
import functools

import jax
import jax.numpy as jnp
from jax import lax
from jax.experimental import pallas as pl
from jax.experimental.pallas import tpu as pltpu
from jax.experimental.pallas import tpu_sc as plsc

N = 10000
E = 320000
D = 128
EPS = 1e-5

NC = 2
NS = 16
NW = NC * NS
CHUNK = 128
NCHUNK = E // CHUNK
ROWS_PER_TILE = 624


def _mesh():
    return plsc.VectorSubcoreMesh(
        core_axis_name="c", subcore_axis_name="s", num_cores=NC,
        num_subcores=NS)


def _sc_segment_sum(recv1d, ef):
    def body(recv_hbm, ef_hbm, out0_hbm, out1_hbm, acc_sh, idx_v, rows_v,
             sem):
        c = lax.axis_index("c")
        s = lax.axis_index("s")
        wid = s * NC + c

        zero16 = jnp.zeros((16,), jnp.float32)

        def zrow(i, _):
            r = i // (D // 16)
            k = i % (D // 16)
            rows_v[r, pl.ds(k * 16, 16)] = zero16
            return 0

        lax.fori_loop(0, CHUNK * (D // 16), zrow, 0)
        base = s * ROWS_PER_TILE
        for i in range(4):
            pltpu.sync_copy(rows_v,
                            acc_sh.at[pl.ds(base + i * CHUNK, CHUNK)])
        pltpu.sync_copy(rows_v.at[pl.ds(0, 112)],
                        acc_sh.at[pl.ds(base + 4 * CHUNK, 112)])

        @pl.when(s == NS - 1)
        def _():
            pltpu.sync_copy(rows_v.at[pl.ds(0, 16)],
                            acc_sh.at[pl.ds(NS * ROWS_PER_TILE, 16)])

        plsc.subcore_barrier()

        def step(j, _):
            g = j * NW + wid

            @pl.when(g < NCHUNK)
            def _():
                pltpu.sync_copy(recv_hbm.at[pl.ds(g * CHUNK, CHUNK)], idx_v)
                pltpu.sync_copy(ef_hbm.at[pl.ds(g * CHUNK, CHUNK)], rows_v)
                pltpu.sync_copy(rows_v, acc_sh.at[idx_v], add=True)

            return 0

        lax.fori_loop(0, (NCHUNK + NW - 1) // NW, step, 0)
        plsc.subcore_barrier()

        @pl.when(c == 0)
        def _():
            pltpu.sync_copy(acc_sh.at[pl.ds(base, ROWS_PER_TILE)],
                            out0_hbm.at[pl.ds(base, ROWS_PER_TILE)])

            @pl.when(s == NS - 1)
            def _():
                pltpu.sync_copy(
                    acc_sh.at[pl.ds(NS * ROWS_PER_TILE, 16)],
                    out0_hbm.at[pl.ds(NS * ROWS_PER_TILE, 16)])

        @pl.when(c == 1)
        def _():
            pltpu.sync_copy(acc_sh.at[pl.ds(base, ROWS_PER_TILE)],
                            out1_hbm.at[pl.ds(base, ROWS_PER_TILE)])

            @pl.when(s == NS - 1)
            def _():
                pltpu.sync_copy(
                    acc_sh.at[pl.ds(NS * ROWS_PER_TILE, 16)],
                    out1_hbm.at[pl.ds(NS * ROWS_PER_TILE, 16)])

    f = pl.kernel(
        body,
        out_type=(jax.ShapeDtypeStruct((N, D), jnp.float32),
                  jax.ShapeDtypeStruct((N, D), jnp.float32)),
        mesh=_mesh(),
        scratch_types=[
            pltpu.VMEM_SHARED((N, D), jnp.float32),
            pltpu.VMEM((CHUNK,), jnp.int32),
            pltpu.VMEM((CHUNK, D), jnp.float32),
            pltpu.SemaphoreType.DMA,
        ],
    )
    return f(recv1d, ef)


def _sc_gather2(nfA, nfB, send1d, recv1d):
    def body(nfA_hbm, nfB_hbm, send_hbm, recv_hbm, outA_hbm, outB_hbm,
             idx_v, rowsA_v, rowsB_v, semA, semB):
        c = lax.axis_index("c")
        s = lax.axis_index("s")
        wid = s * NC + c

        def step(j, _):
            g = j * NW + wid

            @pl.when(g < NCHUNK)
            def _():
                pltpu.sync_copy(send_hbm.at[pl.ds(g * CHUNK, CHUNK)], idx_v)
                cpA = pltpu.async_copy(nfA_hbm.at[idx_v], rowsA_v, semA)
                cpA.wait()
                pltpu.sync_copy(rowsA_v, outA_hbm.at[pl.ds(g * CHUNK, CHUNK)])
                pltpu.sync_copy(recv_hbm.at[pl.ds(g * CHUNK, CHUNK)], idx_v)
                cpB = pltpu.async_copy(nfB_hbm.at[idx_v], rowsB_v, semB)
                cpB.wait()
                pltpu.sync_copy(rowsB_v, outB_hbm.at[pl.ds(g * CHUNK, CHUNK)])

            return 0

        lax.fori_loop(0, (NCHUNK + NW - 1) // NW, step, 0)

    f = pl.kernel(
        body,
        out_type=(jax.ShapeDtypeStruct((E, D), jnp.float32),
                  jax.ShapeDtypeStruct((E, D), jnp.float32)),
        mesh=_mesh(),
        scratch_types=[
            pltpu.VMEM((CHUNK,), jnp.int32),
            pltpu.VMEM((CHUNK, D), jnp.float32),
            pltpu.VMEM((CHUNK, D), jnp.float32),
            pltpu.SemaphoreType.DMA,
            pltpu.SemaphoreType.DMA,
        ],
    )
    return f(nfA, nfB, send1d, recv1d)


def _layer_norm(h, gamma, beta):
    mu = jnp.mean(h, axis=-1, keepdims=True)
    var = jnp.mean((h - mu) ** 2, axis=-1, keepdims=True)
    return (h - mu) * lax.rsqrt(var + EPS) * gamma + beta


def _tc_node_mlp(nodes, agg0, agg1, nW1a, nW1b, nb1, nW2, nb2, ng, nbeta,
                 eW1a, eW1b):
    BN = 1000

    def body(nodes_ref, a0_ref, a1_ref, nW1a_ref, nW1b_ref, nb1_ref,
             nW2_ref, nb2_ref, ng_ref, nbeta_ref, eW1a_ref, eW1b_ref,
             nf_ref, nfA_ref, nfB_ref):
        x = nodes_ref[...]
        a = a0_ref[...] + a1_ref[...]
        h = (jnp.dot(x, nW1a_ref[...], preferred_element_type=jnp.float32)
             + jnp.dot(a, nW1b_ref[...], preferred_element_type=jnp.float32)
             + nb1_ref[...])
        h = jnp.dot(h, nW2_ref[...],
                    preferred_element_type=jnp.float32) + nb2_ref[...]
        nf = _layer_norm(h, ng_ref[...], nbeta_ref[...])
        nf_ref[...] = nf
        nfA_ref[...] = jnp.dot(nf, eW1a_ref[...],
                               preferred_element_type=jnp.float32)
        nfB_ref[...] = jnp.dot(nf, eW1b_ref[...],
                               preferred_element_type=jnp.float32)

    row_spec = pl.BlockSpec((BN, D), lambda i: (i, 0))
    w_spec = pl.BlockSpec((D, D), lambda i: (0, 0))
    v_spec = pl.BlockSpec((D,), lambda i: (0,))
    return pl.pallas_call(
        body,
        grid=(N // BN,),
        in_specs=[row_spec, row_spec, row_spec, w_spec, w_spec, v_spec,
                  w_spec, v_spec, v_spec, v_spec, w_spec, w_spec],
        out_specs=[row_spec, row_spec, row_spec],
        out_shape=[jax.ShapeDtypeStruct((N, D), jnp.float32)] * 3,
    )(nodes, agg0, agg1, nW1a, nW1b, nb1, nW2, nb2, ng, nbeta, eW1a, eW1b)


def _tc_edge_mlp(srcsA, dstsB, ef, eW1c, eb1, eW2, eb2, eg, ebeta):
    BE = 4000

    def body(sA_ref, dB_ref, ef_ref, eW1c_ref, eb1_ref, eW2_ref, eb2_ref,
             eg_ref, ebeta_ref, out_ref):
        h = (sA_ref[...] + dB_ref[...]
             + jnp.dot(ef_ref[...], eW1c_ref[...],
                       preferred_element_type=jnp.float32) + eb1_ref[...])
        h = jnp.dot(h, eW2_ref[...],
                    preferred_element_type=jnp.float32) + eb2_ref[...]
        out_ref[...] = _layer_norm(h, eg_ref[...], ebeta_ref[...])

    row_spec = pl.BlockSpec((BE, D), lambda i: (i, 0))
    w_spec = pl.BlockSpec((D, D), lambda i: (0, 0))
    v_spec = pl.BlockSpec((D,), lambda i: (0,))
    return pl.pallas_call(
        body,
        grid=(E // BE,),
        in_specs=[row_spec, row_spec, row_spec, w_spec, v_spec, w_spec,
                  v_spec, v_spec, v_spec],
        out_specs=row_spec,
        out_shape=jax.ShapeDtypeStruct((E, D), jnp.float32),
    )(srcsA, dstsB, ef, eW1c, eb1, eW2, eb2, eg, ebeta)


def kernel(node_features, edge_features, senders, receivers,
           nW1, nb1, nW2, nb2, ng, nbeta,
           eW1, eb1, eW2, eb2, eg, ebeta):
    nodes = node_features[0]
    ef = edge_features[0]

    agg0, agg1 = _sc_segment_sum(receivers, ef)
    nf, nfA, nfB = _tc_node_mlp(
        nodes, agg0, agg1, nW1[:D], nW1[D:], nb1, nW2, nb2, ng, nbeta,
        eW1[:D], eW1[D:2 * D])
    srcsA, dstsB = _sc_gather2(nfA, nfB, senders, receivers)
    ef_out = _tc_edge_mlp(srcsA, dstsB, ef, eW1[2 * D:], eb1, eW2, eb2,
                          eg, ebeta)
    return (nf[None], ef_out[None])


# speedup vs baseline: 4.6603x; 1.4586x over previous
import functools

import jax
import jax.numpy as jnp
from jax import lax
from jax.experimental import pallas as pl
from jax.experimental.pallas import tpu as pltpu
from jax.experimental.pallas import tpu_sc as plsc

N = 10000
E = 320000
D = 128
EPS = 1e-5

NC = 2
NS = 16
NW = NC * NS
CHUNK = 128
PER_W = 78 * CHUNK
TAIL0 = NW * PER_W
NJ = 79
NB = 3
ROWS_PER_TILE = 624


def _mesh():
    return plsc.VectorSubcoreMesh(
        core_axis_name="c", subcore_axis_name="s", num_cores=NC,
        num_subcores=NS)


def _off(wid, t):
    t = jnp.clip(t, 0, NJ - 1)
    return jnp.where(t < NJ - 1, wid * PER_W + t * CHUNK,
                     TAIL0 + wid * CHUNK)


def _valid(wid, t):
    return (t < NJ - 1) & (t >= 0) | ((t == NJ - 1) & (wid < 4))


def _sc_segment_sum(recv1d, ef):
    def body(recv_hbm, ef_hbm, out0_hbm, out1_hbm, acc_sh, idx_v, rows_v,
             semI, semR, semS):
        c = lax.axis_index("c")
        s = lax.axis_index("s")
        wid = s * NC + c

        zero16 = jnp.zeros((16,), jnp.float32)

        def zrow(i, _):
            r = i // (D // 16)
            k = i % (D // 16)
            rows_v[0, r, pl.ds(k * 16, 16)] = zero16
            return 0

        lax.fori_loop(0, CHUNK * (D // 16), zrow, 0)
        zbase = s * ROWS_PER_TILE
        for i in range(4):
            pltpu.sync_copy(rows_v.at[0],
                            acc_sh.at[pl.ds(zbase + i * CHUNK, CHUNK)])
        pltpu.sync_copy(rows_v.at[0, pl.ds(0, 112)],
                        acc_sh.at[pl.ds(zbase + 4 * CHUNK, 112)])

        @pl.when(s == NS - 1)
        def _():
            pltpu.sync_copy(rows_v.at[0, pl.ds(0, 16)],
                            acc_sh.at[pl.ds(NS * ROWS_PER_TILE, 16)])

        plsc.subcore_barrier()

        def start_loads(t):
            b = t % NB
            o = _off(wid, t)
            pltpu.async_copy(recv_hbm.at[pl.ds(o, CHUNK)], idx_v.at[b],
                             semI)
            pltpu.async_copy(ef_hbm.at[pl.ds(o, CHUNK)], rows_v.at[b],
                             semR)

        def wait_loads(t):
            b = t % NB
            o = _off(wid, t)
            pltpu.make_async_copy(recv_hbm.at[pl.ds(o, CHUNK)],
                                  idx_v.at[b], semI).wait()
            pltpu.make_async_copy(ef_hbm.at[pl.ds(o, CHUNK)],
                                  rows_v.at[b], semR).wait()

        def start_scatter(t):
            b = t % NB
            pltpu.async_copy(rows_v.at[b], acc_sh.at[idx_v.at[b]], semS,
                             add=True)

        def wait_scatter(t):
            b = t % NB
            pltpu.make_async_copy(rows_v.at[b], acc_sh.at[idx_v.at[b]],
                                  semS).wait()

        start_loads(0)

        def step(j, _):
            @pl.when(_valid(wid, j))
            def _():
                wait_loads(j)
                start_scatter(j)

            @pl.when(_valid(wid, j - (NB - 1)))
            def _():
                wait_scatter(j - (NB - 1))

            @pl.when(_valid(wid, j + 1))
            def _():
                start_loads(j + 1)

            return 0

        lax.fori_loop(0, NJ, step, 0)
        for dt in range(NB - 1):
            t = NJ - 1 - dt

            @pl.when(_valid(wid, t))
            def _():
                wait_scatter(t)

        plsc.subcore_barrier()

        def dump(out_hbm):
            pltpu.sync_copy(acc_sh.at[pl.ds(zbase, ROWS_PER_TILE)],
                            out_hbm.at[pl.ds(zbase, ROWS_PER_TILE)])

            @pl.when(s == NS - 1)
            def _():
                pltpu.sync_copy(
                    acc_sh.at[pl.ds(NS * ROWS_PER_TILE, 16)],
                    out_hbm.at[pl.ds(NS * ROWS_PER_TILE, 16)])

        @pl.when(c == 0)
        def _():
            dump(out0_hbm)

        @pl.when(c == 1)
        def _():
            dump(out1_hbm)

    f = pl.kernel(
        body,
        out_type=(jax.ShapeDtypeStruct((N, D), jnp.float32),
                  jax.ShapeDtypeStruct((N, D), jnp.float32)),
        mesh=_mesh(),
        scratch_types=[
            pltpu.VMEM_SHARED((N, D), jnp.float32),
            pltpu.VMEM((NB, CHUNK), jnp.int32),
            pltpu.VMEM((NB, CHUNK, D), jnp.float32),
            pltpu.SemaphoreType.DMA,
            pltpu.SemaphoreType.DMA,
            pltpu.SemaphoreType.DMA,
        ],
    )
    return f(recv1d, ef)


def _sc_gather2(nfA, nfB, send1d, recv1d):
    def body(nfA_hbm, nfB_hbm, send_hbm, recv_hbm, outA_hbm, outB_hbm,
             idxA_v, idxB_v, rowsA_v, rowsB_v,
             semIA, semIB, semGA, semGB, semOA, semOB):
        c = lax.axis_index("c")
        s = lax.axis_index("s")
        wid = s * NC + c

        def start_idx(t):
            b = t % NB
            o = _off(wid, t)
            pltpu.async_copy(send_hbm.at[pl.ds(o, CHUNK)], idxA_v.at[b],
                             semIA)
            pltpu.async_copy(recv_hbm.at[pl.ds(o, CHUNK)], idxB_v.at[b],
                             semIB)

        def wait_idx(t):
            b = t % NB
            o = _off(wid, t)
            pltpu.make_async_copy(send_hbm.at[pl.ds(o, CHUNK)],
                                  idxA_v.at[b], semIA).wait()
            pltpu.make_async_copy(recv_hbm.at[pl.ds(o, CHUNK)],
                                  idxB_v.at[b], semIB).wait()

        def start_gather(t):
            b = t % NB
            pltpu.async_copy(nfA_hbm.at[idxA_v.at[b]], rowsA_v.at[b],
                             semGA)
            pltpu.async_copy(nfB_hbm.at[idxB_v.at[b]], rowsB_v.at[b],
                             semGB)

        def wait_gather(t):
            b = t % NB
            pltpu.make_async_copy(nfA_hbm.at[idxA_v.at[b]],
                                  rowsA_v.at[b], semGA).wait()
            pltpu.make_async_copy(nfB_hbm.at[idxB_v.at[b]],
                                  rowsB_v.at[b], semGB).wait()

        def start_out(t):
            b = t % NB
            o = _off(wid, t)
            pltpu.async_copy(rowsA_v.at[b], outA_hbm.at[pl.ds(o, CHUNK)],
                             semOA)
            pltpu.async_copy(rowsB_v.at[b], outB_hbm.at[pl.ds(o, CHUNK)],
                             semOB)

        def wait_out(t):
            b = t % NB
            o = _off(wid, t)
            pltpu.make_async_copy(rowsA_v.at[b],
                                  outA_hbm.at[pl.ds(o, CHUNK)],
                                  semOA).wait()
            pltpu.make_async_copy(rowsB_v.at[b],
                                  outB_hbm.at[pl.ds(o, CHUNK)],
                                  semOB).wait()

        start_idx(0)

        def step(j, _):
            @pl.when(_valid(wid, j - NB))
            def _():
                wait_out(j - NB)

            @pl.when(_valid(wid, j))
            def _():
                wait_idx(j)
                start_gather(j)

            @pl.when(_valid(wid, j + 1))
            def _():
                start_idx(j + 1)

            @pl.when(_valid(wid, j))
            def _():
                wait_gather(j)
                start_out(j)

            return 0

        lax.fori_loop(0, NJ, step, 0)
        for dt in range(NB):
            t = NJ - 1 - dt

            @pl.when(_valid(wid, t))
            def _():
                wait_out(t)

    f = pl.kernel(
        body,
        out_type=(jax.ShapeDtypeStruct((E, D), jnp.float32),
                  jax.ShapeDtypeStruct((E, D), jnp.float32)),
        mesh=_mesh(),
        scratch_types=[
            pltpu.VMEM((NB, CHUNK), jnp.int32),
            pltpu.VMEM((NB, CHUNK), jnp.int32),
            pltpu.VMEM((NB, CHUNK, D), jnp.float32),
            pltpu.VMEM((NB, CHUNK, D), jnp.float32),
            pltpu.SemaphoreType.DMA,
            pltpu.SemaphoreType.DMA,
            pltpu.SemaphoreType.DMA,
            pltpu.SemaphoreType.DMA,
            pltpu.SemaphoreType.DMA,
            pltpu.SemaphoreType.DMA,
        ],
    )
    return f(nfA, nfB, send1d, recv1d)


def _layer_norm(h, gamma, beta):
    mu = jnp.mean(h, axis=-1, keepdims=True)
    var = jnp.mean((h - mu) ** 2, axis=-1, keepdims=True)
    return (h - mu) * lax.rsqrt(var + EPS) * gamma + beta


def _tc_node_mlp(nodes, agg0, agg1, nW1a, nW1b, nb1, nW2, nb2, ng, nbeta,
                 eW1a, eW1b):
    BN = 1000

    def body(nodes_ref, a0_ref, a1_ref, nW1a_ref, nW1b_ref, nb1_ref,
             nW2_ref, nb2_ref, ng_ref, nbeta_ref, eW1a_ref, eW1b_ref,
             nf_ref, nfA_ref, nfB_ref):
        x = nodes_ref[...]
        a = a0_ref[...] + a1_ref[...]
        h = (jnp.dot(x, nW1a_ref[...], preferred_element_type=jnp.float32)
             + jnp.dot(a, nW1b_ref[...], preferred_element_type=jnp.float32)
             + nb1_ref[...])
        h = jnp.dot(h, nW2_ref[...],
                    preferred_element_type=jnp.float32) + nb2_ref[...]
        nf = _layer_norm(h, ng_ref[...], nbeta_ref[...])
        nf_ref[...] = nf
        nfA_ref[...] = jnp.dot(nf, eW1a_ref[...],
                               preferred_element_type=jnp.float32)
        nfB_ref[...] = jnp.dot(nf, eW1b_ref[...],
                               preferred_element_type=jnp.float32)

    row_spec = pl.BlockSpec((BN, D), lambda i: (i, 0))
    w_spec = pl.BlockSpec((D, D), lambda i: (0, 0))
    v_spec = pl.BlockSpec((D,), lambda i: (0,))
    return pl.pallas_call(
        body,
        grid=(N // BN,),
        in_specs=[row_spec, row_spec, row_spec, w_spec, w_spec, v_spec,
                  w_spec, v_spec, v_spec, v_spec, w_spec, w_spec],
        out_specs=[row_spec, row_spec, row_spec],
        out_shape=[jax.ShapeDtypeStruct((N, D), jnp.float32)] * 3,
    )(nodes, agg0, agg1, nW1a, nW1b, nb1, nW2, nb2, ng, nbeta, eW1a, eW1b)


def _tc_edge_mlp(srcsA, dstsB, ef, eW1c, eb1, eW2, eb2, eg, ebeta):
    BE = 4000

    def body(sA_ref, dB_ref, ef_ref, eW1c_ref, eb1_ref, eW2_ref, eb2_ref,
             eg_ref, ebeta_ref, out_ref):
        h = (sA_ref[...] + dB_ref[...]
             + jnp.dot(ef_ref[...], eW1c_ref[...],
                       preferred_element_type=jnp.float32) + eb1_ref[...])
        h = jnp.dot(h, eW2_ref[...],
                    preferred_element_type=jnp.float32) + eb2_ref[...]
        out_ref[...] = _layer_norm(h, eg_ref[...], ebeta_ref[...])

    row_spec = pl.BlockSpec((BE, D), lambda i: (i, 0))
    w_spec = pl.BlockSpec((D, D), lambda i: (0, 0))
    v_spec = pl.BlockSpec((D,), lambda i: (0,))
    return pl.pallas_call(
        body,
        grid=(E // BE,),
        in_specs=[row_spec, row_spec, row_spec, w_spec, v_spec, w_spec,
                  v_spec, v_spec, v_spec],
        out_specs=row_spec,
        out_shape=jax.ShapeDtypeStruct((E, D), jnp.float32),
    )(srcsA, dstsB, ef, eW1c, eb1, eW2, eb2, eg, ebeta)


def kernel(node_features, edge_features, senders, receivers,
           nW1, nb1, nW2, nb2, ng, nbeta,
           eW1, eb1, eW2, eb2, eg, ebeta):
    nodes = node_features[0]
    ef = edge_features[0]

    agg0, agg1 = _sc_segment_sum(receivers, ef)
    nf, nfA, nfB = _tc_node_mlp(
        nodes, agg0, agg1, nW1[:D], nW1[D:], nb1, nW2, nb2, ng, nbeta,
        eW1[:D], eW1[D:2 * D])
    srcsA, dstsB = _sc_gather2(nfA, nfB, senders, receivers)
    ef_out = _tc_edge_mlp(srcsA, dstsB, ef, eW1[2 * D:], eb1, eW2, eb2,
                          eg, ebeta)
    return (nf[None], ef_out[None])
